# baseline (speedup 1.0000x reference)
import functools

import jax
import jax.numpy as jnp
from jax import lax
from jax.experimental import pallas as pl
from jax.experimental.pallas import tpu as pltpu
from jax.experimental.pallas import tpu_sc as plsc

N_ROWS = 16384
N_COLS = 1000
LANES = 16
N_WORKERS = 32
ROWS_PER_WORKER = N_ROWS // N_WORKERS
G = 16
N_PAIRS = ROWS_PER_WORKER // (2 * G)
NEG = -10.0 ** 10
MARGIN_FY = 0.8
MARGIN_FNY = 0.3
N_CHUNKS = N_COLS // LANES


def _sc_body(x_hbm, lbl_hbm, out_hbm, xbuf, lblbuf, ostage, sem0, sem1):
    wid = lax.axis_index("c") * 16 + lax.axis_index("s")
    row0 = wid * ROWS_PER_WORKER

    pltpu.sync_copy(lbl_hbm.at[pl.ds(row0, ROWS_PER_WORKER)], lblbuf)

    def dma(g, slot, sem):
        src = x_hbm.at[pl.ds(row0 + g * G, G), :]
        dst = xbuf.at[pl.ds(slot * G, G), :]
        return pltpu.make_async_copy(src, dst, sem)

    dma(0, 0, sem0).start()
    dma(1, 1, sem1).start()

    lane = lax.iota(jnp.int32, LANES)

    def process(g, slot, sem, acc):
        dma(g, slot, sem).wait()
        labels = lblbuf[pl.ds(g * G, G)]
        rowidx = slot * G + lane
        fy = plsc.load_gather(xbuf, [rowidx, labels])
        plsc.store_scatter(xbuf, [rowidx, labels],
                           jnp.full((LANES,), NEG, jnp.float32))

        def row_max(r, m_vec):
            row = slot * G + r
            chunks = [xbuf[row, pl.ds(LANES * j, LANES)]
                      for j in range(N_CHUNKS)]
            chunks.append(xbuf[row, pl.ds(N_COLS - LANES, LANES)])
            while len(chunks) > 1:
                nxt = [jnp.maximum(chunks[i], chunks[i + 1])
                       for i in range(0, len(chunks) - 1, 2)]
                if len(chunks) % 2:
                    nxt.append(chunks[-1])
                chunks = nxt
            m_r = jnp.max(chunks[0])
            return jnp.where(lane == r, m_r, m_vec)

        m = lax.fori_loop(0, G, row_max, jnp.full((LANES,), NEG, jnp.float32))
        l = jnp.maximum(MARGIN_FY - fy, 0.0) + jnp.maximum(MARGIN_FNY + m, 0.0)
        return acc + l

    def pair(g2, acc):
        g0 = 2 * g2
        acc = process(g0, 0, sem0, acc)

        @pl.when(g2 < N_PAIRS - 1)
        def _():
            dma(g0 + 2, 0, sem0).start()

        acc = process(g0 + 1, 1, sem1, acc)

        @pl.when(g2 < N_PAIRS - 1)
        def _():
            dma(g0 + 3, 1, sem1).start()

        return acc

    acc = lax.fori_loop(0, N_PAIRS, pair, jnp.zeros((LANES,), jnp.float32))
    ostage[...] = acc * (1.0 / N_ROWS)
    pltpu.sync_copy(ostage, out_hbm.at[wid])


_sc_loss = functools.partial(
    pl.kernel,
    out_type=jax.ShapeDtypeStruct((N_WORKERS, LANES), jnp.float32),
    mesh=plsc.VectorSubcoreMesh(core_axis_name="c", subcore_axis_name="s"),
    compiler_params=pltpu.CompilerParams(needs_layout_passes=False,
                                         use_tc_tiling_on_sc=True),
    scratch_types=[
        pltpu.VMEM((2 * G, N_COLS), jnp.float32),
        pltpu.VMEM((ROWS_PER_WORKER,), jnp.int32),
        pltpu.VMEM((LANES,), jnp.float32),
        pltpu.SemaphoreType.DMA,
        pltpu.SemaphoreType.DMA,
    ],
)(_sc_body)


def kernel(x, label):
    parts = _sc_loss(x, label.astype(jnp.int32))
    return jnp.sum(parts)


# speedup vs baseline: 2.3111x; 2.3111x over previous
import functools

import jax
import jax.numpy as jnp
from jax import lax
from jax.experimental import pallas as pl
from jax.experimental.pallas import tpu as pltpu
from jax.experimental.pallas import tpu_sc as plsc

N_SAMPLES = 16384
N_CLASSES = 1000
LANES = 16
N_WORKERS = 32
SPW = N_SAMPLES // N_WORKERS
SB = 128
N_BLOCKS = SPW // SB
SUBS = SB // LANES
CHUNK = 128
ROWS = [CHUNK] * 7 + [N_CLASSES - 7 * CHUNK]
N_CHUNKS = len(ROWS)
N_SLOTS = 4
NEG = -10.0 ** 10
MARGIN_FY = 0.8
MARGIN_FNY = 0.3


def _sc_body(xt_hbm, lbl_hbm, out_hbm, xbuf, lblbuf, ostage, *sems):
    wid = lax.axis_index("c") * 16 + lax.axis_index("s")
    s0 = wid * SPW

    pltpu.sync_copy(lbl_hbm.at[pl.ds(s0, SPW)], lblbuf)

    lane = lax.iota(jnp.int32, LANES)

    def dma(b, k, sem):
        slot = k % N_SLOTS
        src = xt_hbm.at[pl.ds(CHUNK * k, ROWS[k]), pl.ds(s0 + b * SB, SB)]
        dst = xbuf.at[slot, pl.ds(0, ROWS[k]), :]
        return pltpu.make_async_copy(src, dst, sem)

    for k in range(N_SLOTS - 1):
        dma(0, k, sems[k % N_SLOTS]).start()

    def block_body(b, acc):
        m = [jnp.full((LANES,), NEG, jnp.float32) for _ in range(SUBS)]
        fy = [jnp.zeros((LANES,), jnp.float32) for _ in range(SUBS)]

        for k in range(N_CHUNKS):
            rows_k = ROWS[k]
            slot = k % N_SLOTS
            sem = sems[slot]
            dma(b, k, sem).wait()

            pre = k + N_SLOTS - 1
            if pre < N_CHUNKS:
                dma(b, pre, sems[pre % N_SLOTS]).start()
            else:
                pk = pre - N_CHUNKS

                @pl.when(b + 1 < N_BLOCKS)
                def _():
                    dma(b + 1, pk, sems[pk % N_SLOTS]).start()

            slot_vec = jnp.full((LANES,), slot, jnp.int32)
            neg_vec = jnp.full((LANES,), NEG, jnp.float32)
            for sub in range(SUBS):
                lbl_sub = lblbuf[pl.ds(b * SB + sub * LANES, LANES)]
                rel = lbl_sub - CHUNK * k
                mask = (rel >= 0) & (rel < rows_k)
                relc = jnp.clip(rel, 0, rows_k - 1)
                got = plsc.load_gather(
                    xbuf, [slot_vec, relc, sub * LANES + lane], mask=mask)
                fy[sub] = jnp.where(mask, got, fy[sub])
                plsc.store_scatter(
                    xbuf, [slot_vec, relc, sub * LANES + lane], neg_vec,
                    mask=mask)

            def row_body(ri, ms):
                r = 2 * ri
                ms = tuple(
                    jnp.maximum(ms[i], xbuf[slot, r, pl.ds(LANES * i, LANES)])
                    for i in range(SUBS))
                return tuple(
                    jnp.maximum(ms[i],
                                xbuf[slot, r + 1, pl.ds(LANES * i, LANES)])
                    for i in range(SUBS))

            m = list(lax.fori_loop(0, rows_k // 2, row_body, tuple(m)))

        for sub in range(SUBS):
            acc = acc + (jnp.maximum(MARGIN_FY - fy[sub], 0.0)
                         + jnp.maximum(MARGIN_FNY + m[sub], 0.0))
        return acc

    acc = lax.fori_loop(0, N_BLOCKS, block_body,
                        jnp.zeros((LANES,), jnp.float32))
    ostage[...] = acc * (1.0 / N_SAMPLES)
    pltpu.sync_copy(ostage, out_hbm.at[wid])


_sc_loss = functools.partial(
    pl.kernel,
    out_type=jax.ShapeDtypeStruct((N_WORKERS, LANES), jnp.float32),
    mesh=plsc.VectorSubcoreMesh(core_axis_name="c", subcore_axis_name="s"),
    compiler_params=pltpu.CompilerParams(needs_layout_passes=False,
                                         use_tc_tiling_on_sc=True),
    scratch_types=[
        pltpu.VMEM((N_SLOTS, CHUNK, SB), jnp.float32),
        pltpu.VMEM((SPW,), jnp.int32),
        pltpu.VMEM((LANES,), jnp.float32),
    ] + [pltpu.SemaphoreType.DMA] * N_SLOTS,
)(_sc_body)


def kernel(x, label):
    parts = _sc_loss(x.T, label.astype(jnp.int32))
    return jnp.sum(parts)
